# Initial kernel scaffold; baseline (speedup 1.0000x reference)
import jax
import jax.numpy as jnp
from jax.experimental import pallas as pl


def kernel(x, emb, W_ih, W_hh, b_ih, b_hh, fc_w, fc_b):
    raise NotImplementedError("write your pallas kernel here")


# speedup vs baseline: 3.3875x; 3.3875x over previous
import functools

import jax
import jax.numpy as jnp
from jax import lax
from jax.experimental import pallas as pl
from jax.experimental.pallas import tpu as pltpu
from jax.experimental.pallas import tpu_sc as plsc

EMBED = 128
HIDDEN = 64
B = 1024
T = 200

_NC = 2
_NS = 16
_NW = _NC * _NS
_ROWS = B * T
_RPW = _ROWS // _NW
_CHUNK = 128
_NCHUNK = _RPW // _CHUNK


def _sc_gather(table, idx_flat):
    mesh = plsc.VectorSubcoreMesh(core_axis_name="c", subcore_axis_name="s")

    @functools.partial(
        pl.kernel,
        mesh=mesh,
        out_type=jax.ShapeDtypeStruct((_ROWS, EMBED), jnp.float32),
        scratch_types=[
            pltpu.VMEM((_CHUNK,), jnp.int32),
            pltpu.VMEM((_CHUNK, EMBED), jnp.float32),
            pltpu.SemaphoreType.DMA,
        ],
    )
    def gather_kernel(table_hbm, idx_hbm, out_hbm, idx_v, rows_v, sem):
        wid = lax.axis_index("s") * _NC + lax.axis_index("c")
        base = wid * _RPW

        def body(i, carry):
            off = pl.multiple_of(base + i * _CHUNK, _CHUNK)
            pltpu.sync_copy(idx_hbm.at[pl.ds(off, _CHUNK)], idx_v)
            pltpu.async_copy(table_hbm.at[idx_v], rows_v, sem).wait()
            pltpu.sync_copy(rows_v, out_hbm.at[pl.ds(off, _CHUNK)])
            return carry

        lax.fori_loop(0, _NCHUNK, body, 0)

    return gather_kernel(table, idx_flat)


def _gru_body(e_ref, wih_ref, whh_ref, bih_ref, bhh_ref, fcw_ref, fcb_ref,
              out_ref, h_scr):
    t = pl.program_id(0)

    @pl.when(t == 0)
    def _init():
        h_scr[...] = jnp.zeros_like(h_scr)

    e = e_ref[0]
    h = h_scr[...]
    gi = jnp.dot(e, wih_ref[...], preferred_element_type=jnp.float32) + bih_ref[...]
    gh = jnp.dot(h, whh_ref[...], preferred_element_type=jnp.float32) + bhh_ref[...]
    r = jax.nn.sigmoid(gi[:, :HIDDEN] + gh[:, :HIDDEN])
    z = jax.nn.sigmoid(gi[:, HIDDEN:2 * HIDDEN] + gh[:, HIDDEN:2 * HIDDEN])
    n = jnp.tanh(gi[:, 2 * HIDDEN:] + r * gh[:, 2 * HIDDEN:])
    h_new = (1.0 - z) * n + z * h
    h_scr[...] = h_new

    @pl.when(t == T - 1)
    def _finish():
        logits = jnp.sum(h_new * fcw_ref[...], axis=1, keepdims=True) + fcb_ref[...]
        out_ref[...] = jax.nn.sigmoid(logits)


def _tc_gru(e3, wih_t, whh_t, bih2, bhh2, fcw, fcb2):
    return pl.pallas_call(
        _gru_body,
        grid=(T,),
        in_specs=[
            pl.BlockSpec((1, B, EMBED), lambda t: (t, 0, 0)),
            pl.BlockSpec((EMBED, 3 * HIDDEN), lambda t: (0, 0)),
            pl.BlockSpec((HIDDEN, 3 * HIDDEN), lambda t: (0, 0)),
            pl.BlockSpec((1, 3 * HIDDEN), lambda t: (0, 0)),
            pl.BlockSpec((1, 3 * HIDDEN), lambda t: (0, 0)),
            pl.BlockSpec((1, HIDDEN), lambda t: (0, 0)),
            pl.BlockSpec((1, 1), lambda t: (0, 0)),
        ],
        out_specs=pl.BlockSpec((B, 1), lambda t: (0, 0)),
        out_shape=jax.ShapeDtypeStruct((B, 1), jnp.float32),
        scratch_shapes=[pltpu.VMEM((B, HIDDEN), jnp.float32)],
    )(e3, wih_t, whh_t, bih2, bhh2, fcw, fcb2)


def kernel(x, emb, W_ih, W_hh, b_ih, b_hh, fc_w, fc_b):
    idx = x.astype(jnp.int32).T.reshape(-1)
    e = _sc_gather(emb, idx)
    e3 = e.reshape(T, B, EMBED)
    return _tc_gru(e3, W_ih.T, W_hh.T,
                   b_ih.reshape(1, -1), b_hh.reshape(1, -1),
                   fc_w, fc_b.reshape(1, 1))


# speedup vs baseline: 4.1589x; 1.2277x over previous
import functools

import jax
import jax.numpy as jnp
from jax import lax
from jax.experimental import pallas as pl
from jax.experimental.pallas import tpu as pltpu
from jax.experimental.pallas import tpu_sc as plsc

EMBED = 128
HIDDEN = 64
B = 1024
T = 200

_NC = 2
_NS = 16
_NW = _NC * _NS
_ROWS = B * T
_RPW = _ROWS // _NW
_CHUNK = 128
_NCHUNK = _RPW // _CHUNK


def _sc_gather(table, idx_flat):
    mesh = plsc.VectorSubcoreMesh(core_axis_name="c", subcore_axis_name="s")
    idx3 = idx_flat.reshape(_NW, _NCHUNK, _CHUNK)

    @functools.partial(
        pl.kernel,
        mesh=mesh,
        out_type=jax.ShapeDtypeStruct((_ROWS, EMBED), jnp.float32),
        scratch_types=[
            pltpu.VMEM((_NCHUNK, _CHUNK), jnp.int32),
            pltpu.VMEM((_CHUNK, EMBED), jnp.float32),
            pltpu.VMEM((_CHUNK, EMBED), jnp.float32),
            pltpu.SemaphoreType.DMA,
            pltpu.SemaphoreType.DMA,
        ],
    )
    def gather_kernel(table_hbm, idx_hbm, out_hbm, idx_v, r0, r1, s0, s1):
        bufs = (r0, r1)
        sems = (s0, s1)
        wid = lax.axis_index("s") * _NC + lax.axis_index("c")
        base = wid * _RPW
        pltpu.sync_copy(idx_hbm.at[wid], idx_v)

        def start(i, b):
            pltpu.async_copy(table_hbm.at[idx_v.at[i]], bufs[b], sems[b])

        def drain(b):
            pltpu.make_async_copy(table_hbm.at[idx_v.at[0]], bufs[b], sems[b]).wait()

        for b in range(2):
            start(b, b)

        def body(g, carry):
            for b in range(2):
                i = g * 2 + b
                drain(b)
                off = pl.multiple_of(base + i * _CHUNK, _CHUNK)
                pltpu.sync_copy(bufs[b], out_hbm.at[pl.ds(off, _CHUNK)])

                @pl.when(i + 2 < _NCHUNK)
                def _refill():
                    start(i + 2, b)

            return carry

        lax.fori_loop(0, _NCHUNK // 2, body, 0)

    return gather_kernel(table, idx3)


def _gru_body(e_ref, wih_ref, whh_ref, bih_ref, bhh_ref, fcw_ref, fcb_ref,
              out_ref, h_scr):
    t = pl.program_id(0)

    @pl.when(t == 0)
    def _init():
        h_scr[...] = jnp.zeros_like(h_scr)

    e = e_ref[0]
    h = h_scr[...]
    gi = jnp.dot(e, wih_ref[...], preferred_element_type=jnp.float32) + bih_ref[...]
    gh = jnp.dot(h, whh_ref[...], preferred_element_type=jnp.float32) + bhh_ref[...]
    r = jax.nn.sigmoid(gi[:, :HIDDEN] + gh[:, :HIDDEN])
    z = jax.nn.sigmoid(gi[:, HIDDEN:2 * HIDDEN] + gh[:, HIDDEN:2 * HIDDEN])
    n = jnp.tanh(gi[:, 2 * HIDDEN:] + r * gh[:, 2 * HIDDEN:])
    h_new = (1.0 - z) * n + z * h
    h_scr[...] = h_new

    @pl.when(t == T - 1)
    def _finish():
        logits = jnp.sum(h_new * fcw_ref[...], axis=1, keepdims=True) + fcb_ref[...]
        out_ref[...] = jax.nn.sigmoid(logits)


def _tc_gru(e3, wih_t, whh_t, bih2, bhh2, fcw, fcb2):
    return pl.pallas_call(
        _gru_body,
        grid=(T,),
        in_specs=[
            pl.BlockSpec((1, B, EMBED), lambda t: (t, 0, 0)),
            pl.BlockSpec((EMBED, 3 * HIDDEN), lambda t: (0, 0)),
            pl.BlockSpec((HIDDEN, 3 * HIDDEN), lambda t: (0, 0)),
            pl.BlockSpec((1, 3 * HIDDEN), lambda t: (0, 0)),
            pl.BlockSpec((1, 3 * HIDDEN), lambda t: (0, 0)),
            pl.BlockSpec((1, HIDDEN), lambda t: (0, 0)),
            pl.BlockSpec((1, 1), lambda t: (0, 0)),
        ],
        out_specs=pl.BlockSpec((B, 1), lambda t: (0, 0)),
        out_shape=jax.ShapeDtypeStruct((B, 1), jnp.float32),
        scratch_shapes=[pltpu.VMEM((B, HIDDEN), jnp.float32)],
    )(e3, wih_t, whh_t, bih2, bhh2, fcw, fcb2)


def kernel(x, emb, W_ih, W_hh, b_ih, b_hh, fc_w, fc_b):
    idx = x.astype(jnp.int32).T.reshape(-1)
    e = _sc_gather(emb, idx)
    e3 = e.reshape(T, B, EMBED)
    return _tc_gru(e3, W_ih.T, W_hh.T,
                   b_ih.reshape(1, -1), b_hh.reshape(1, -1),
                   fc_w, fc_b.reshape(1, 1))


# speedup vs baseline: 4.4845x; 1.0783x over previous
import functools

import jax
import jax.numpy as jnp
from jax import lax
from jax.experimental import pallas as pl
from jax.experimental.pallas import tpu as pltpu
from jax.experimental.pallas import tpu_sc as plsc

EMBED = 128
HIDDEN = 64
B = 1024
T = 200

_NSEG = 5
_TSEG = T // _NSEG

_NC = 2
_NS = 16
_NW = _NC * _NS
_CHUNK = 128


def _sc_gather(table, idx3, rows, nchunk):
    rpw = rows // _NW
    mesh = plsc.VectorSubcoreMesh(core_axis_name="c", subcore_axis_name="s")

    @functools.partial(
        pl.kernel,
        mesh=mesh,
        out_type=jax.ShapeDtypeStruct((rows, EMBED), jnp.float32),
        scratch_types=[
            pltpu.VMEM((nchunk, _CHUNK), jnp.int32),
            pltpu.VMEM((_CHUNK, EMBED), jnp.float32),
            pltpu.VMEM((_CHUNK, EMBED), jnp.float32),
            pltpu.SemaphoreType.DMA,
            pltpu.SemaphoreType.DMA,
        ],
    )
    def gather_kernel(table_hbm, idx_hbm, out_hbm, idx_v, r0, r1, s0, s1):
        bufs = (r0, r1)
        sems = (s0, s1)
        wid = lax.axis_index("s") * _NC + lax.axis_index("c")
        base = wid * rpw
        pltpu.sync_copy(idx_hbm.at[wid], idx_v)

        def start(i, b):
            pltpu.async_copy(table_hbm.at[idx_v.at[i]], bufs[b], sems[b])

        def drain(b):
            pltpu.make_async_copy(table_hbm.at[idx_v.at[0]], bufs[b], sems[b]).wait()

        for b in range(2):
            start(b, b)

        def body(g, carry):
            for b in range(2):
                i = g * 2 + b
                drain(b)
                off = pl.multiple_of(base + i * _CHUNK, _CHUNK)
                pltpu.sync_copy(bufs[b], out_hbm.at[pl.ds(off, _CHUNK)])

                @pl.when(i + 2 < nchunk)
                def _refill():
                    start(i + 2, b)

            return carry

        lax.fori_loop(0, nchunk // 2, body, 0)

    return gather_kernel(table, idx3)


def _gru_body(e_ref, h0_ref, wih_ref, whh_ref, bih_ref, bhh_ref, fcw_ref,
              fcb_ref, hout_ref, y_ref, h_scr):
    t = pl.program_id(0)

    @pl.when(t == 0)
    def _init():
        h_scr[...] = h0_ref[...]

    e = e_ref[0]
    h = h_scr[...]
    gi = jnp.dot(e, wih_ref[...], preferred_element_type=jnp.float32) + bih_ref[...]
    gh = jnp.dot(h, whh_ref[...], preferred_element_type=jnp.float32) + bhh_ref[...]
    r = jax.nn.sigmoid(gi[:, :HIDDEN] + gh[:, :HIDDEN])
    z = jax.nn.sigmoid(gi[:, HIDDEN:2 * HIDDEN] + gh[:, HIDDEN:2 * HIDDEN])
    n = jnp.tanh(gi[:, 2 * HIDDEN:] + r * gh[:, 2 * HIDDEN:])
    h_new = (1.0 - z) * n + z * h
    h_scr[...] = h_new

    @pl.when(t == _TSEG - 1)
    def _finish():
        hout_ref[...] = h_new
        logits = jnp.sum(h_new * fcw_ref[...], axis=1, keepdims=True) + fcb_ref[...]
        y_ref[...] = jax.nn.sigmoid(logits)


def _tc_gru_seg(e3, h0, wih_t, whh_t, bih2, bhh2, fcw, fcb2):
    return pl.pallas_call(
        _gru_body,
        grid=(_TSEG,),
        in_specs=[
            pl.BlockSpec((1, B, EMBED), lambda t: (t, 0, 0)),
            pl.BlockSpec((B, HIDDEN), lambda t: (0, 0)),
            pl.BlockSpec((EMBED, 3 * HIDDEN), lambda t: (0, 0)),
            pl.BlockSpec((HIDDEN, 3 * HIDDEN), lambda t: (0, 0)),
            pl.BlockSpec((1, 3 * HIDDEN), lambda t: (0, 0)),
            pl.BlockSpec((1, 3 * HIDDEN), lambda t: (0, 0)),
            pl.BlockSpec((1, HIDDEN), lambda t: (0, 0)),
            pl.BlockSpec((1, 1), lambda t: (0, 0)),
        ],
        out_specs=[
            pl.BlockSpec((B, HIDDEN), lambda t: (0, 0)),
            pl.BlockSpec((B, 1), lambda t: (0, 0)),
        ],
        out_shape=[
            jax.ShapeDtypeStruct((B, HIDDEN), jnp.float32),
            jax.ShapeDtypeStruct((B, 1), jnp.float32),
        ],
        scratch_shapes=[pltpu.VMEM((B, HIDDEN), jnp.float32)],
    )(e3, h0, wih_t, whh_t, bih2, bhh2, fcw, fcb2)


def kernel(x, emb, W_ih, W_hh, b_ih, b_hh, fc_w, fc_b):
    idx = x.astype(jnp.int32).T.reshape(_NSEG, _TSEG * B)
    seg_rows = _TSEG * B
    nchunk = seg_rows // (_NW * _CHUNK)

    wih_t = W_ih.T
    whh_t = W_hh.T
    bih2 = b_ih.reshape(1, -1)
    bhh2 = b_hh.reshape(1, -1)
    fcb2 = fc_b.reshape(1, 1)

    h = jnp.zeros((B, HIDDEN), jnp.float32)
    y = None
    for s in range(_NSEG):
        idx3 = idx[s].reshape(_NW, nchunk, _CHUNK)
        e = _sc_gather(emb, idx3, seg_rows, nchunk)
        e3 = e.reshape(_TSEG, B, EMBED)
        h, y = _tc_gru_seg(e3, h, wih_t, whh_t, bih2, bhh2, fc_w, fcb2)
    return y


# speedup vs baseline: 4.7825x; 1.0665x over previous
import functools

import jax
import jax.numpy as jnp
from jax import lax
from jax.experimental import pallas as pl
from jax.experimental.pallas import tpu as pltpu
from jax.experimental.pallas import tpu_sc as plsc

EMBED = 128
HIDDEN = 64
B = 1024
T = 200

_NSEG = 5
_TSEG = T // _NSEG

_NC = 2
_NS = 16
_NW = _NC * _NS
_CHUNK = 128


def _sc_gather(table, idx3, rows, nchunk):
    rpw = rows // _NW
    mesh = plsc.VectorSubcoreMesh(core_axis_name="c", subcore_axis_name="s")

    @functools.partial(
        pl.kernel,
        mesh=mesh,
        out_type=jax.ShapeDtypeStruct((rows, EMBED), jnp.float32),
        scratch_types=[
            pltpu.VMEM((nchunk, _CHUNK), jnp.int32),
            pltpu.VMEM((_CHUNK, EMBED), jnp.float32),
            pltpu.VMEM((_CHUNK, EMBED), jnp.float32),
            pltpu.SemaphoreType.DMA,
            pltpu.SemaphoreType.DMA,
        ],
    )
    def gather_kernel(table_hbm, idx_hbm, out_hbm, idx_v, r0, r1, s0, s1):
        bufs = (r0, r1)
        sems = (s0, s1)
        wid = lax.axis_index("s") * _NC + lax.axis_index("c")
        base = wid * rpw
        pltpu.sync_copy(idx_hbm.at[wid], idx_v)

        def start(i, b):
            pltpu.async_copy(table_hbm.at[idx_v.at[i]], bufs[b], sems[b])

        def drain(b):
            pltpu.make_async_copy(table_hbm.at[idx_v.at[0]], bufs[b], sems[b]).wait()

        for b in range(2):
            start(b, b)

        def body(g, carry):
            for b in range(2):
                i = g * 2 + b
                drain(b)
                off = pl.multiple_of(base + i * _CHUNK, _CHUNK)
                pltpu.sync_copy(bufs[b], out_hbm.at[pl.ds(off, _CHUNK)])

                @pl.when(i + 2 < nchunk)
                def _refill():
                    start(i + 2, b)

            return carry

        lax.fori_loop(0, nchunk // 2, body, 0)

    return gather_kernel(table, idx3)


def _gru_body(e_ref, h0_ref, wih_ref, whh_ref, bih_ref, bhh_ref, fcw_ref,
              fcb_ref, hout_ref, y_ref, h_scr):
    t = pl.program_id(0)

    @pl.when(t == 0)
    def _init():
        h_scr[...] = h0_ref[...]

    e = e_ref[0]
    h = h_scr[...]
    gi = lax.dot_general(wih_ref[...], e, (((1,), (1,)), ((), ())),
                         preferred_element_type=jnp.float32) + bih_ref[...]
    gh = jnp.dot(whh_ref[...], h, preferred_element_type=jnp.float32) + bhh_ref[...]
    r = jax.nn.sigmoid(gi[:HIDDEN] + gh[:HIDDEN])
    z = jax.nn.sigmoid(gi[HIDDEN:2 * HIDDEN] + gh[HIDDEN:2 * HIDDEN])
    n = jnp.tanh(gi[2 * HIDDEN:] + r * gh[2 * HIDDEN:])
    h_new = (1.0 - z) * n + z * h
    h_scr[...] = h_new

    @pl.when(t == _TSEG - 1)
    def _finish():
        hout_ref[...] = h_new
        logits = jnp.sum(h_new * fcw_ref[...], axis=0, keepdims=True) + fcb_ref[...]
        y_ref[...] = jax.nn.sigmoid(logits)


def _tc_gru_seg(e3, h0, wih, whh, bih2, bhh2, fcwT, fcb2):
    return pl.pallas_call(
        _gru_body,
        grid=(_TSEG,),
        in_specs=[
            pl.BlockSpec((1, B, EMBED), lambda t: (t, 0, 0)),
            pl.BlockSpec((HIDDEN, B), lambda t: (0, 0)),
            pl.BlockSpec((3 * HIDDEN, EMBED), lambda t: (0, 0)),
            pl.BlockSpec((3 * HIDDEN, HIDDEN), lambda t: (0, 0)),
            pl.BlockSpec((3 * HIDDEN, 1), lambda t: (0, 0)),
            pl.BlockSpec((3 * HIDDEN, 1), lambda t: (0, 0)),
            pl.BlockSpec((HIDDEN, 1), lambda t: (0, 0)),
            pl.BlockSpec((1, 1), lambda t: (0, 0)),
        ],
        out_specs=[
            pl.BlockSpec((HIDDEN, B), lambda t: (0, 0)),
            pl.BlockSpec((1, B), lambda t: (0, 0)),
        ],
        out_shape=[
            jax.ShapeDtypeStruct((HIDDEN, B), jnp.float32),
            jax.ShapeDtypeStruct((1, B), jnp.float32),
        ],
        scratch_shapes=[pltpu.VMEM((HIDDEN, B), jnp.float32)],
    )(e3, h0, wih, whh, bih2, bhh2, fcwT, fcb2)


def kernel(x, emb, W_ih, W_hh, b_ih, b_hh, fc_w, fc_b):
    idx = x.astype(jnp.int32).T.reshape(_NSEG, _TSEG * B)
    seg_rows = _TSEG * B
    nchunk = seg_rows // (_NW * _CHUNK)

    bih2 = b_ih.reshape(-1, 1)
    bhh2 = b_hh.reshape(-1, 1)
    fcwT = fc_w.reshape(-1, 1)
    fcb2 = fc_b.reshape(1, 1)

    h = jnp.zeros((HIDDEN, B), jnp.float32)
    y = None
    for s in range(_NSEG):
        idx3 = idx[s].reshape(_NW, nchunk, _CHUNK)
        e = _sc_gather(emb, idx3, seg_rows, nchunk)
        e3 = e.reshape(_TSEG, B, EMBED)
        h, y = _tc_gru_seg(e3, h, W_ih, W_hh, bih2, bhh2, fcwT, fcb2)
    return y.reshape(B, 1)


# speedup vs baseline: 6.2427x; 1.3053x over previous
import functools

import jax
import jax.numpy as jnp
from jax import lax
from jax.experimental import pallas as pl
from jax.experimental.pallas import tpu as pltpu
from jax.experimental.pallas import tpu_sc as plsc

EMBED = 128
HIDDEN = 64
B = 1024
T = 200

_NSEG = 5
_TSEG = T // _NSEG

_NC = 2
_NS = 16
_NW = _NC * _NS
_CHUNK = 128


def _sc_gather(table, idx3, rows, nchunk):
    rpw = rows // _NW
    mesh = plsc.VectorSubcoreMesh(core_axis_name="c", subcore_axis_name="s")

    @functools.partial(
        pl.kernel,
        mesh=mesh,
        out_type=jax.ShapeDtypeStruct((rows, EMBED), jnp.float32),
        scratch_types=[
            pltpu.VMEM((nchunk, _CHUNK), jnp.int32),
            pltpu.VMEM((_CHUNK, EMBED), jnp.float32),
            pltpu.VMEM((_CHUNK, EMBED), jnp.float32),
            pltpu.SemaphoreType.DMA,
            pltpu.SemaphoreType.DMA,
        ],
    )
    def gather_kernel(table_hbm, idx_hbm, out_hbm, idx_v, r0, r1, s0, s1):
        bufs = (r0, r1)
        sems = (s0, s1)
        wid = lax.axis_index("s") * _NC + lax.axis_index("c")
        base = wid * rpw
        pltpu.sync_copy(idx_hbm.at[wid], idx_v)

        def start(i, b):
            pltpu.async_copy(table_hbm.at[idx_v.at[i]], bufs[b], sems[b])

        def drain(b):
            pltpu.make_async_copy(table_hbm.at[idx_v.at[0]], bufs[b], sems[b]).wait()

        for b in range(2):
            start(b, b)

        def body(g, carry):
            for b in range(2):
                i = g * 2 + b
                drain(b)
                off = pl.multiple_of(base + i * _CHUNK, _CHUNK)
                pltpu.sync_copy(bufs[b], out_hbm.at[pl.ds(off, _CHUNK)])

                @pl.when(i + 2 < nchunk)
                def _refill():
                    start(i + 2, b)

            return carry

        lax.fori_loop(0, nchunk // 2, body, 0)

    return gather_kernel(table, idx3)


_NT = 2


def _gru_body(e_ref, h0_ref, wih_ref, whh_ref, bih_ref, bhh_ref, fcw_ref,
              fcb_ref, hout_ref, y_ref, h_scr):
    t = pl.program_id(0)

    @pl.when(t == 0)
    def _init():
        h_scr[...] = h0_ref[...]

    h = h_scr[...]
    for j in range(_NT):
        e = e_ref[j]
        gi = lax.dot_general(wih_ref[...], e, (((1,), (1,)), ((), ())),
                             preferred_element_type=jnp.float32) + bih_ref[...]
        gh = jnp.dot(whh_ref[...], h, preferred_element_type=jnp.float32) + bhh_ref[...]
        r = jax.nn.sigmoid(gi[:HIDDEN] + gh[:HIDDEN])
        z = jax.nn.sigmoid(gi[HIDDEN:2 * HIDDEN] + gh[HIDDEN:2 * HIDDEN])
        n = jnp.tanh(gi[2 * HIDDEN:] + r * gh[2 * HIDDEN:])
        h = (1.0 - z) * n + z * h
    h_scr[...] = h

    @pl.when(t == _TSEG // _NT - 1)
    def _finish():
        hout_ref[...] = h
        logits = jnp.sum(h * fcw_ref[...], axis=0, keepdims=True) + fcb_ref[...]
        y_ref[...] = jax.nn.sigmoid(logits)


def _tc_gru_seg(e3, h0, wih, whh, bih2, bhh2, fcwT, fcb2):
    return pl.pallas_call(
        _gru_body,
        grid=(_TSEG // _NT,),
        in_specs=[
            pl.BlockSpec((_NT, B, EMBED), lambda t: (t, 0, 0)),
            pl.BlockSpec((HIDDEN, B), lambda t: (0, 0)),
            pl.BlockSpec((3 * HIDDEN, EMBED), lambda t: (0, 0)),
            pl.BlockSpec((3 * HIDDEN, HIDDEN), lambda t: (0, 0)),
            pl.BlockSpec((3 * HIDDEN, 1), lambda t: (0, 0)),
            pl.BlockSpec((3 * HIDDEN, 1), lambda t: (0, 0)),
            pl.BlockSpec((HIDDEN, 1), lambda t: (0, 0)),
            pl.BlockSpec((1, 1), lambda t: (0, 0)),
        ],
        out_specs=[
            pl.BlockSpec((HIDDEN, B), lambda t: (0, 0)),
            pl.BlockSpec((1, B), lambda t: (0, 0)),
        ],
        out_shape=[
            jax.ShapeDtypeStruct((HIDDEN, B), jnp.float32),
            jax.ShapeDtypeStruct((1, B), jnp.float32),
        ],
        scratch_shapes=[pltpu.VMEM((HIDDEN, B), jnp.float32)],
    )(e3, h0, wih, whh, bih2, bhh2, fcwT, fcb2)


def kernel(x, emb, W_ih, W_hh, b_ih, b_hh, fc_w, fc_b):
    idx = x.astype(jnp.int32).T.reshape(_NSEG, _TSEG * B)
    seg_rows = _TSEG * B
    nchunk = seg_rows // (_NW * _CHUNK)

    bih2 = b_ih.reshape(-1, 1)
    bhh2 = b_hh.reshape(-1, 1)
    fcwT = fc_w.reshape(-1, 1)
    fcb2 = fc_b.reshape(1, 1)

    h = jnp.zeros((HIDDEN, B), jnp.float32)
    y = None
    for s in range(_NSEG):
        idx3 = idx[s].reshape(_NW, nchunk, _CHUNK)
        e = _sc_gather(emb, idx3, seg_rows, nchunk)
        e3 = e.reshape(_TSEG, B, EMBED)
        h, y = _tc_gru_seg(e3, h, W_ih, W_hh, bih2, bhh2, fcwT, fcb2)
    return y.reshape(B, 1)


# speedup vs baseline: 7.2570x; 1.1625x over previous
import functools

import jax
import jax.numpy as jnp
from jax import lax
from jax.experimental import pallas as pl
from jax.experimental.pallas import tpu as pltpu
from jax.experimental.pallas import tpu_sc as plsc

EMBED = 128
HIDDEN = 64
B = 1024
T = 200

_NSEG = 5
_TSEG = T // _NSEG

_NC = 2
_NS = 16
_NW = _NC * _NS
_CHUNK = 128


def _sc_gather(table, idx3, rows, nchunk):
    rpw = rows // _NW
    mesh = plsc.VectorSubcoreMesh(core_axis_name="c", subcore_axis_name="s")

    @functools.partial(
        pl.kernel,
        mesh=mesh,
        out_type=jax.ShapeDtypeStruct((rows, EMBED), jnp.float32),
        scratch_types=[
            pltpu.VMEM((nchunk, _CHUNK), jnp.int32),
            pltpu.VMEM((_CHUNK, EMBED), jnp.float32),
            pltpu.VMEM((_CHUNK, EMBED), jnp.float32),
            pltpu.SemaphoreType.DMA,
            pltpu.SemaphoreType.DMA,
        ],
    )
    def gather_kernel(table_hbm, idx_hbm, out_hbm, idx_v, r0, r1, s0, s1):
        bufs = (r0, r1)
        sems = (s0, s1)
        wid = lax.axis_index("s") * _NC + lax.axis_index("c")
        base = wid * rpw
        pltpu.sync_copy(idx_hbm.at[wid], idx_v)

        def start(i, b):
            pltpu.async_copy(table_hbm.at[idx_v.at[i]], bufs[b], sems[b])

        def drain(b):
            pltpu.make_async_copy(table_hbm.at[idx_v.at[0]], bufs[b], sems[b]).wait()

        for b in range(2):
            start(b, b)

        def body(g, carry):
            for b in range(2):
                i = g * 2 + b
                drain(b)
                off = pl.multiple_of(base + i * _CHUNK, _CHUNK)
                pltpu.sync_copy(bufs[b], out_hbm.at[pl.ds(off, _CHUNK)])

                @pl.when(i + 2 < nchunk)
                def _refill():
                    start(i + 2, b)

            return carry

        lax.fori_loop(0, nchunk // 2, body, 0)

    return gather_kernel(table, idx3)


_NT = 4


def _gru_body(e_ref, h0_ref, wih_ref, whh_ref, bih_ref, bhh_ref, fcw_ref,
              fcb_ref, hout_ref, y_ref, h_scr):
    t = pl.program_id(0)

    @pl.when(t == 0)
    def _init():
        h_scr[...] = h0_ref[...]

    h = h_scr[...]
    for j in range(_NT):
        e = e_ref[j]
        gi = lax.dot_general(wih_ref[...], e, (((1,), (1,)), ((), ())),
                             preferred_element_type=jnp.float32) + bih_ref[...]
        gh = jnp.dot(whh_ref[...], h, preferred_element_type=jnp.float32) + bhh_ref[...]
        r = jax.nn.sigmoid(gi[:HIDDEN] + gh[:HIDDEN])
        z = jax.nn.sigmoid(gi[HIDDEN:2 * HIDDEN] + gh[HIDDEN:2 * HIDDEN])
        n = jnp.tanh(gi[2 * HIDDEN:] + r * gh[2 * HIDDEN:])
        h = (1.0 - z) * n + z * h
    h_scr[...] = h

    @pl.when(t == _TSEG // _NT - 1)
    def _finish():
        hout_ref[...] = h
        logits = jnp.sum(h * fcw_ref[...], axis=0, keepdims=True) + fcb_ref[...]
        y_ref[...] = jax.nn.sigmoid(logits)


def _tc_gru_seg(e3, h0, wih, whh, bih2, bhh2, fcwT, fcb2):
    return pl.pallas_call(
        _gru_body,
        grid=(_TSEG // _NT,),
        in_specs=[
            pl.BlockSpec((_NT, B, EMBED), lambda t: (t, 0, 0)),
            pl.BlockSpec((HIDDEN, B), lambda t: (0, 0)),
            pl.BlockSpec((3 * HIDDEN, EMBED), lambda t: (0, 0)),
            pl.BlockSpec((3 * HIDDEN, HIDDEN), lambda t: (0, 0)),
            pl.BlockSpec((3 * HIDDEN, 1), lambda t: (0, 0)),
            pl.BlockSpec((3 * HIDDEN, 1), lambda t: (0, 0)),
            pl.BlockSpec((HIDDEN, 1), lambda t: (0, 0)),
            pl.BlockSpec((1, 1), lambda t: (0, 0)),
        ],
        out_specs=[
            pl.BlockSpec((HIDDEN, B), lambda t: (0, 0)),
            pl.BlockSpec((1, B), lambda t: (0, 0)),
        ],
        out_shape=[
            jax.ShapeDtypeStruct((HIDDEN, B), jnp.float32),
            jax.ShapeDtypeStruct((1, B), jnp.float32),
        ],
        scratch_shapes=[pltpu.VMEM((HIDDEN, B), jnp.float32)],
    )(e3, h0, wih, whh, bih2, bhh2, fcwT, fcb2)


def kernel(x, emb, W_ih, W_hh, b_ih, b_hh, fc_w, fc_b):
    idx = x.astype(jnp.int32).T.reshape(_NSEG, _TSEG * B)
    seg_rows = _TSEG * B
    nchunk = seg_rows // (_NW * _CHUNK)

    bih2 = b_ih.reshape(-1, 1)
    bhh2 = b_hh.reshape(-1, 1)
    fcwT = fc_w.reshape(-1, 1)
    fcb2 = fc_b.reshape(1, 1)

    h = jnp.zeros((HIDDEN, B), jnp.float32)
    y = None
    for s in range(_NSEG):
        idx3 = idx[s].reshape(_NW, nchunk, _CHUNK)
        e = _sc_gather(emb, idx3, seg_rows, nchunk)
        e3 = e.reshape(_TSEG, B, EMBED)
        h, y = _tc_gru_seg(e3, h, W_ih, W_hh, bih2, bhh2, fcwT, fcb2)
    return y.reshape(B, 1)


# speedup vs baseline: 7.4970x; 1.0331x over previous
import functools

import jax
import jax.numpy as jnp
from jax import lax
from jax.experimental import pallas as pl
from jax.experimental.pallas import tpu as pltpu
from jax.experimental.pallas import tpu_sc as plsc

EMBED = 128
HIDDEN = 64
B = 1024
T = 200

_NSEG = 5
_TSEG = T // _NSEG

_NC = 2
_NS = 16
_NW = _NC * _NS
_CHUNK = 128


def _sc_gather(table, idx3, rows, nchunk):
    rpw = rows // _NW
    mesh = plsc.VectorSubcoreMesh(core_axis_name="c", subcore_axis_name="s")

    @functools.partial(
        pl.kernel,
        mesh=mesh,
        out_type=jax.ShapeDtypeStruct((rows, EMBED), jnp.float32),
        scratch_types=[
            pltpu.VMEM((nchunk, _CHUNK), jnp.int32),
            pltpu.VMEM((_CHUNK, EMBED), jnp.float32),
            pltpu.VMEM((_CHUNK, EMBED), jnp.float32),
            pltpu.SemaphoreType.DMA,
            pltpu.SemaphoreType.DMA,
        ],
    )
    def gather_kernel(table_hbm, idx_hbm, out_hbm, idx_v, r0, r1, s0, s1):
        bufs = (r0, r1)
        sems = (s0, s1)
        wid = lax.axis_index("s") * _NC + lax.axis_index("c")
        base = wid * rpw
        pltpu.sync_copy(idx_hbm.at[wid], idx_v)

        def start(i, b):
            pltpu.async_copy(table_hbm.at[idx_v.at[i]], bufs[b], sems[b])

        def drain(b):
            pltpu.make_async_copy(table_hbm.at[idx_v.at[0]], bufs[b], sems[b]).wait()

        for b in range(2):
            start(b, b)

        def body(g, carry):
            for b in range(2):
                i = g * 2 + b
                drain(b)
                off = pl.multiple_of(base + i * _CHUNK, _CHUNK)
                pltpu.sync_copy(bufs[b], out_hbm.at[pl.ds(off, _CHUNK)])

                @pl.when(i + 2 < nchunk)
                def _refill():
                    start(i + 2, b)

            return carry

        lax.fori_loop(0, nchunk // 2, body, 0)

    return gather_kernel(table, idx3)


_NT = 8


def _gru_body(e_ref, h0_ref, wih_ref, whh_ref, bih_ref, bhh_ref, fcw_ref,
              fcb_ref, hout_ref, y_ref, h_scr):
    t = pl.program_id(0)

    @pl.when(t == 0)
    def _init():
        h_scr[...] = h0_ref[...]

    h = h_scr[...]
    for j in range(_NT):
        e = e_ref[j]
        gi = lax.dot_general(wih_ref[...], e, (((1,), (1,)), ((), ())),
                             preferred_element_type=jnp.float32) + bih_ref[...]
        gh = jnp.dot(whh_ref[...], h, preferred_element_type=jnp.float32) + bhh_ref[...]
        r = jax.nn.sigmoid(gi[:HIDDEN] + gh[:HIDDEN])
        z = jax.nn.sigmoid(gi[HIDDEN:2 * HIDDEN] + gh[HIDDEN:2 * HIDDEN])
        n = jnp.tanh(gi[2 * HIDDEN:] + r * gh[2 * HIDDEN:])
        h = (1.0 - z) * n + z * h
    h_scr[...] = h

    @pl.when(t == _TSEG // _NT - 1)
    def _finish():
        hout_ref[...] = h
        logits = jnp.sum(h * fcw_ref[...], axis=0, keepdims=True) + fcb_ref[...]
        y_ref[...] = jax.nn.sigmoid(logits)


def _tc_gru_seg(e3, h0, wih, whh, bih2, bhh2, fcwT, fcb2):
    return pl.pallas_call(
        _gru_body,
        grid=(_TSEG // _NT,),
        in_specs=[
            pl.BlockSpec((_NT, B, EMBED), lambda t: (t, 0, 0)),
            pl.BlockSpec((HIDDEN, B), lambda t: (0, 0)),
            pl.BlockSpec((3 * HIDDEN, EMBED), lambda t: (0, 0)),
            pl.BlockSpec((3 * HIDDEN, HIDDEN), lambda t: (0, 0)),
            pl.BlockSpec((3 * HIDDEN, 1), lambda t: (0, 0)),
            pl.BlockSpec((3 * HIDDEN, 1), lambda t: (0, 0)),
            pl.BlockSpec((HIDDEN, 1), lambda t: (0, 0)),
            pl.BlockSpec((1, 1), lambda t: (0, 0)),
        ],
        out_specs=[
            pl.BlockSpec((HIDDEN, B), lambda t: (0, 0)),
            pl.BlockSpec((1, B), lambda t: (0, 0)),
        ],
        out_shape=[
            jax.ShapeDtypeStruct((HIDDEN, B), jnp.float32),
            jax.ShapeDtypeStruct((1, B), jnp.float32),
        ],
        scratch_shapes=[pltpu.VMEM((HIDDEN, B), jnp.float32)],
    )(e3, h0, wih, whh, bih2, bhh2, fcwT, fcb2)


def kernel(x, emb, W_ih, W_hh, b_ih, b_hh, fc_w, fc_b):
    idx = x.astype(jnp.int32).T.reshape(_NSEG, _TSEG * B)
    seg_rows = _TSEG * B
    nchunk = seg_rows // (_NW * _CHUNK)

    bih2 = b_ih.reshape(-1, 1)
    bhh2 = b_hh.reshape(-1, 1)
    fcwT = fc_w.reshape(-1, 1)
    fcb2 = fc_b.reshape(1, 1)

    h = jnp.zeros((HIDDEN, B), jnp.float32)
    y = None
    for s in range(_NSEG):
        idx3 = idx[s].reshape(_NW, nchunk, _CHUNK)
        e = _sc_gather(emb, idx3, seg_rows, nchunk)
        e3 = e.reshape(_TSEG, B, EMBED)
        h, y = _tc_gru_seg(e3, h, W_ih, W_hh, bih2, bhh2, fcwT, fcb2)
    return y.reshape(B, 1)
